# baseline (speedup 1.0000x reference)
import functools
import math

import jax
import jax.numpy as jnp
from jax import lax
from jax.experimental import pallas as pl
from jax.experimental.pallas import tpu as pltpu
from jax.experimental.pallas import tpu_sc as plsc

_IDX_W = 128
_CHUNK_ROWS = 512
_R = _CHUNK_ROWS // _IDX_W


@functools.cache
def _build(N, V, D, n_cores, n_subcores):
    nw = n_cores * n_subcores
    rows_per_worker = N // nw
    n_chunks = rows_per_worker // _CHUNK_ROWS
    scale = float(math.sqrt(D))
    mesh = plsc.VectorSubcoreMesh(core_axis_name="c", subcore_axis_name="s")

    @functools.partial(
        pl.kernel,
        mesh=mesh,
        out_type=jax.ShapeDtypeStruct((N, D), jnp.float32),
        scratch_types=[
            pltpu.VMEM((_R, _IDX_W), jnp.int32),
            pltpu.VMEM((_CHUNK_ROWS, D), jnp.float32),
            pltpu.SemaphoreType.DMA,
        ],
        compiler_params=pltpu.CompilerParams(use_tc_tiling_on_sc=False),
    )
    def gather_kernel(x_hbm, table_hbm, out_hbm, idx_v, rows_v, sem):
        wid = lax.axis_index("s") * n_cores + lax.axis_index("c")
        chunk0 = wid * n_chunks

        def chunk_body(j, carry):
            row0 = (chunk0 + j) * _CHUNK_ROWS
            irow0 = (chunk0 + j) * _R
            pltpu.sync_copy(x_hbm.at[pl.ds(irow0, _R)], idx_v)
            copies = [
                pltpu.async_copy(
                    table_hbm.at[idx_v.at[r]],
                    rows_v.at[pl.ds(r * _IDX_W, _IDX_W)],
                    sem,
                )
                for r in range(_R)
            ]
            for cp in copies:
                cp.wait()

            def scale_body(i, c):
                for k in range(D // 16):
                    sl = pl.ds(k * 16, 16)
                    rows_v[i, sl] = rows_v[i, sl] * scale
                return c

            lax.fori_loop(0, _CHUNK_ROWS, scale_body, 0)
            pltpu.sync_copy(rows_v, out_hbm.at[pl.ds(row0, _CHUNK_ROWS)])
            return carry

        lax.fori_loop(0, n_chunks, chunk_body, 0)

    return gather_kernel


def kernel(x, table):
    B, L = x.shape
    V, D = table.shape
    N = B * L
    info = plsc.get_sparse_core_info()
    fn = _build(N, V, D, info.num_cores, info.num_subcores)
    x2d = x.reshape(N // _IDX_W, _IDX_W)
    out = fn(x2d, table)
    return out.reshape(B, L, D)


# speedup vs baseline: 1.0894x; 1.0894x over previous
import functools
import math

import jax
import jax.numpy as jnp
from jax import lax
from jax.experimental import pallas as pl
from jax.experimental.pallas import tpu as pltpu
from jax.experimental.pallas import tpu_sc as plsc

_IDX_W = 128
_CHUNK_ROWS = 512
_R = _CHUNK_ROWS // _IDX_W


@functools.cache
def _build(N, V, D, n_cores, n_subcores):
    nw = n_cores * n_subcores
    rows_per_worker = N // nw
    n_chunks = rows_per_worker // _CHUNK_ROWS
    n_pairs = n_chunks // 2
    scale = float(math.sqrt(D))
    mesh = plsc.VectorSubcoreMesh(core_axis_name="c", subcore_axis_name="s")

    @functools.partial(
        pl.kernel,
        mesh=mesh,
        out_type=jax.ShapeDtypeStruct((N, D), jnp.float32),
        scratch_types=[
            pltpu.VMEM((2, _R, _IDX_W), jnp.int32),
            pltpu.VMEM((2, _CHUNK_ROWS, D), jnp.float32),
            pltpu.SemaphoreType.DMA,
            pltpu.SemaphoreType.DMA,
            pltpu.SemaphoreType.DMA,
            pltpu.SemaphoreType.DMA,
        ],
        compiler_params=pltpu.CompilerParams(use_tc_tiling_on_sc=False),
    )
    def gather_kernel(x_hbm, table_hbm, out_hbm, idx_v, rows_v, g0, g1, o0, o1):
        wid = lax.axis_index("s") * n_cores + lax.axis_index("c")
        chunk0 = wid * n_chunks
        gsems = (g0, g1)
        osems = (o0, o1)

        def fire_chunk(j, b):
            pltpu.sync_copy(x_hbm.at[pl.ds(j * _R, _R)], idx_v.at[b])
            for r in range(_R):
                pltpu.async_copy(
                    table_hbm.at[idx_v.at[b, r]],
                    rows_v.at[b, pl.ds(r * _IDX_W, _IDX_W)],
                    gsems[b],
                )

        def wait_chunk(b):
            for r in range(_R):
                pltpu.make_async_copy(
                    table_hbm.at[idx_v.at[b, r]],
                    rows_v.at[b, pl.ds(r * _IDX_W, _IDX_W)],
                    gsems[b],
                ).wait()

        def scale_chunk(b):
            def body(i, c):
                for k in range(D // 16):
                    sl = pl.ds(k * 16, 16)
                    rows_v[b, i, sl] = rows_v[b, i, sl] * scale
                return c

            lax.fori_loop(0, _CHUNK_ROWS, body, 0)

        def fire_out(j, b):
            pltpu.async_copy(
                rows_v.at[b],
                out_hbm.at[pl.ds(j * _CHUNK_ROWS, _CHUNK_ROWS)],
                osems[b],
            )

        def drain_out(j, b):
            pltpu.make_async_copy(
                rows_v.at[b],
                out_hbm.at[pl.ds(j * _CHUNK_ROWS, _CHUNK_ROWS)],
                osems[b],
            ).wait()

        fire_chunk(chunk0, 0)
        fire_chunk(chunk0 + 1, 1)

        def pair_body(jj, carry):
            c0 = chunk0 + 2 * jj
            c1 = c0 + 1
            wait_chunk(0)
            scale_chunk(0)
            fire_out(c0, 0)
            wait_chunk(1)
            scale_chunk(1)
            fire_out(c1, 1)
            drain_out(c0, 0)
            fire_chunk(c0 + 2, 0)
            drain_out(c1, 1)
            fire_chunk(c1 + 2, 1)
            return carry

        lax.fori_loop(0, n_pairs - 1, pair_body, 0)

        l0 = chunk0 + 2 * (n_pairs - 1)
        wait_chunk(0)
        scale_chunk(0)
        fire_out(l0, 0)
        wait_chunk(1)
        scale_chunk(1)
        fire_out(l0 + 1, 1)
        drain_out(l0, 0)
        drain_out(l0 + 1, 1)

    return gather_kernel


def kernel(x, table):
    B, L = x.shape
    V, D = table.shape
    N = B * L
    info = plsc.get_sparse_core_info()
    fn = _build(N, V, D, info.num_cores, info.num_subcores)
    x2d = x.reshape(N // _IDX_W, _IDX_W)
    out = fn(x2d, table)
    return out.reshape(B, L, D)


# speedup vs baseline: 1.3077x; 1.2004x over previous
import functools
import math

import jax
import jax.numpy as jnp
from jax import lax
from jax.experimental import pallas as pl
from jax.experimental.pallas import tpu as pltpu
from jax.experimental.pallas import tpu_sc as plsc

_IDX_W = 128
_CHUNK_ROWS = 128
_R = _CHUNK_ROWS // _IDX_W
_PAD_D = 128


@functools.cache
def _build(N, V, D, n_cores, n_subcores):
    nw = n_cores * n_subcores
    rows_per_worker = N // nw
    n_chunks = rows_per_worker // _CHUNK_ROWS
    n_pairs = n_chunks // 2
    scale = float(math.sqrt(D))
    mesh = plsc.VectorSubcoreMesh(core_axis_name="c", subcore_axis_name="s")

    @functools.partial(
        pl.kernel,
        mesh=mesh,
        out_type=jax.ShapeDtypeStruct((N, D), jnp.float32),
        scratch_types=[
            pltpu.VMEM((2, _R, _IDX_W), jnp.int32),
            pltpu.VMEM((2, _CHUNK_ROWS, _PAD_D), jnp.float32),
            pltpu.VMEM((2, _CHUNK_ROWS, D), jnp.float32),
            pltpu.SemaphoreType.DMA,
            pltpu.SemaphoreType.DMA,
            pltpu.SemaphoreType.DMA,
            pltpu.SemaphoreType.DMA,
        ],
        compiler_params=pltpu.CompilerParams(use_tc_tiling_on_sc=True),
    )
    def gather_kernel(
        x_hbm, tpad_hbm, out_hbm, idx_v, rows_v, outv, g0, g1, o0, o1
    ):
        wid = lax.axis_index("s") * n_cores + lax.axis_index("c")
        chunk0 = wid * n_chunks
        gsems = (g0, g1)
        osems = (o0, o1)

        def fire_chunk(j, b):
            pltpu.sync_copy(x_hbm.at[pl.ds(j * _R, _R)], idx_v.at[b])
            for r in range(_R):
                pltpu.async_copy(
                    tpad_hbm.at[idx_v.at[b, r]],
                    rows_v.at[b, pl.ds(r * _IDX_W, _IDX_W)],
                    gsems[b],
                )

        def wait_chunk(b):
            for r in range(_R):
                pltpu.make_async_copy(
                    tpad_hbm.at[idx_v.at[b, r]],
                    rows_v.at[b, pl.ds(r * _IDX_W, _IDX_W)],
                    gsems[b],
                ).wait()

        def scale_chunk(b):
            def body(i, c):
                for k in range(D // 16):
                    sl = pl.ds(k * 16, 16)
                    outv[b, i, sl] = rows_v[b, i, sl] * scale
                return c

            lax.fori_loop(0, _CHUNK_ROWS, body, 0)

        def fire_out(j, b):
            pltpu.async_copy(
                outv.at[b],
                out_hbm.at[pl.ds(j * _CHUNK_ROWS, _CHUNK_ROWS)],
                osems[b],
            )

        def drain_out(j, b):
            pltpu.make_async_copy(
                outv.at[b],
                out_hbm.at[pl.ds(j * _CHUNK_ROWS, _CHUNK_ROWS)],
                osems[b],
            ).wait()

        fire_chunk(chunk0, 0)
        fire_chunk(chunk0 + 1, 1)

        def pair_body(jj, carry):
            c0 = chunk0 + 2 * jj
            c1 = c0 + 1
            wait_chunk(0)
            scale_chunk(0)
            fire_out(c0, 0)
            wait_chunk(1)
            scale_chunk(1)
            fire_out(c1, 1)
            drain_out(c0, 0)
            fire_chunk(c0 + 2, 0)
            drain_out(c1, 1)
            fire_chunk(c1 + 2, 1)
            return carry

        lax.fori_loop(0, n_pairs - 1, pair_body, 0)

        l0 = chunk0 + 2 * (n_pairs - 1)
        wait_chunk(0)
        scale_chunk(0)
        fire_out(l0, 0)
        wait_chunk(1)
        scale_chunk(1)
        fire_out(l0 + 1, 1)
        drain_out(l0, 0)
        drain_out(l0 + 1, 1)

    return gather_kernel


def kernel(x, table):
    B, L = x.shape
    V, D = table.shape
    N = B * L
    info = plsc.get_sparse_core_info()
    fn = _build(N, V, D, info.num_cores, info.num_subcores)
    x2d = x.reshape(N // _IDX_W, _IDX_W)
    tpad = jnp.pad(table, ((0, 0), (0, _PAD_D - D)))
    out = fn(x2d, tpad)
    return out.reshape(B, L, D)
